# baseline (speedup 1.0000x reference)
import functools

import jax
import jax.numpy as jnp
from jax.experimental import pallas as pl
from jax.experimental.pallas import tpu as pltpu


def _maxpool3x3s2(a, sel):
    H, W, C = a.shape
    Ho, Wo = (H - 3) // 2 + 1, (W - 3) // 2 + 1
    v = _relu_bf16(a)
    Hp = ((H + 1) // 2) * 2
    if Hp != H:
        v = jnp.concatenate([v, jnp.zeros((Hp - H, W, C), v.dtype)], axis=0)
    v = v.reshape(Hp // 2, 2, W, C)
    e0, e1 = v[:, 0], v[:, 1]
    rm = jnp.maximum(jnp.maximum(e0[:Ho], e1[:Ho]), e0[1:Ho + 1])
    sm = jnp.maximum(
        jnp.maximum(rm[:, 0:W - 2], rm[:, 1:W - 1]), rm[:, 2:W])
    selb = jnp.broadcast_to(sel[None], (Ho, Wo, W - 2))
    out = jax.lax.dot_general(
        selb, sm, (((2,), (1,)), ((0,), (0,))),
        preferred_element_type=jnp.float32)
    return out.astype(jnp.bfloat16)


def _zpad(v, p):
    H, W, C = v.shape
    v = jnp.concatenate(
        [jnp.zeros((H, p, C), v.dtype), v, jnp.zeros((H, p, C), v.dtype)], axis=1)
    Wp = W + 2 * p
    v = jnp.concatenate(
        [jnp.zeros((p, Wp, C), v.dtype), v, jnp.zeros((p, Wp, C), v.dtype)], axis=0)
    return v


def _rowcat(v, kw, wo):
    return jnp.concatenate([v[:, dw:dw + wo, :] for dw in range(kw)], axis=-1)


def _conv_taps(xw, wr, ho, kh):
    acc = None
    for dh in range(kh):
        a = jax.lax.dot_general(
            xw[dh:dh + ho], wr[dh],
            (((2,), (0,)), ((), ())), preferred_element_type=jnp.float32)
        acc = a if acc is None else acc + a
    return acc


def _relu_bf16(a):
    return jnp.maximum(a, 0.0).astype(jnp.bfloat16)


def _net_kernel(p1_ref, w1_ref, w2_ref, w3_ref, w4_ref, w5_ref,
                sel1_ref, sel2_ref, sel3_ref, o_ref):
    a = jax.lax.dot_general(
        p1_ref[0], w1_ref[...],
        (((2,), (0,)), ((), ())), preferred_element_type=jnp.float32)
    x = _maxpool3x3s2(a, sel1_ref[...])

    xw = _rowcat(_zpad(x, 2), 5, 103)
    x = _maxpool3x3s2(_conv_taps(xw, w2_ref[...], 27, 5),
                      sel2_ref[...])

    xw = _rowcat(_zpad(x, 1), 3, 51)
    x = _relu_bf16(_conv_taps(xw, w3_ref[...], 13, 3))

    xw = _rowcat(_zpad(x, 1), 3, 51)
    x = _relu_bf16(_conv_taps(xw, w4_ref[...], 13, 3))

    xw = _rowcat(_zpad(x, 1), 3, 51)
    o_ref[0] = _maxpool3x3s2(_conv_taps(xw, w5_ref[...], 13, 3),
                             sel3_ref[...])


def _fc1_kernel(a_ref, w_ref, b_ref, o_ref, acc_ref):
    k = pl.program_id(1)

    @pl.when(k == 0)
    def _():
        acc_ref[...] = jnp.zeros_like(acc_ref)

    acc_ref[...] += jnp.dot(a_ref[...], w_ref[...],
                            preferred_element_type=jnp.float32)

    @pl.when(k == pl.num_programs(1) - 1)
    def _():
        o_ref[...] = _relu_bf16(acc_ref[...] + b_ref[...])


def _head_kernel(a_ref, w2_ref, b2_ref, w3_ref, b3_ref, o_ref):
    h = jnp.dot(a_ref[...], w2_ref[...], preferred_element_type=jnp.float32)
    h = _relu_bf16(h + b2_ref[...])
    z = jnp.dot(h, w3_ref[...], preferred_element_type=jnp.float32)
    z = jnp.maximum(z + b3_ref[...], 0.0)
    col = jax.lax.broadcasted_iota(jnp.int32, z.shape, 1)
    mask = col < 10
    zm = jnp.where(mask, z, -jnp.inf)
    m = jnp.max(zm, axis=1, keepdims=True)
    e = jnp.where(mask, jnp.exp(zm - m), 0.0)
    s = jnp.sum(e, axis=1, keepdims=True)
    o_ref[...] = e / s


def kernel(x, c1, c2, c3, c4, c5, l1_w, l1_b, l2_w, l2_b, l3_w, l3_b):
    N = x.shape[0]
    xb = x[:, 0].astype(jnp.bfloat16)

    patches1 = jax.lax.conv_general_dilated_patches(
        xb[..., None], (11, 11), (2, 2), "VALID",
        dimension_numbers=("NHWC", "HWIO", "NHWC"))
    patches1 = jnp.pad(patches1, ((0, 0), (0, 0), (0, 0), (0, 7)))
    c1p = jnp.pad(c1, ((0, 7), (0, 0)))

    w2r = c2.reshape(96, 5, 5, 256).transpose(1, 2, 0, 3).reshape(5, 480, 256)
    w3r = c3.reshape(256, 3, 3, 384).transpose(1, 2, 0, 3).reshape(3, 768, 384)
    w4r = c4.reshape(384, 3, 3, 384).transpose(1, 2, 0, 3).reshape(3, 1152, 384)
    w5r = c5.reshape(384, 3, 3, 256).transpose(1, 2, 0, 3).reshape(3, 1152, 256)

    def _sel(wo, wm):
        return (jnp.arange(wo)[:, None] * 2
                == jnp.arange(wm)[None, :]).astype(jnp.bfloat16)
    sel1, sel2, sel3 = _sel(103, 205), _sel(51, 101), _sel(25, 49)

    feats = pl.pallas_call(
        _net_kernel,
        out_shape=jax.ShapeDtypeStruct((N, 6, 25, 256), jnp.bfloat16),
        grid=(N,),
        in_specs=[
            pl.BlockSpec((1, 55, 207, 128), lambda n: (n, 0, 0, 0)),
            pl.BlockSpec((128, 96), lambda n: (0, 0)),
            pl.BlockSpec((5, 480, 256), lambda n: (0, 0, 0)),
            pl.BlockSpec((3, 768, 384), lambda n: (0, 0, 0)),
            pl.BlockSpec((3, 1152, 384), lambda n: (0, 0, 0)),
            pl.BlockSpec((3, 1152, 256), lambda n: (0, 0, 0)),
            pl.BlockSpec((103, 205), lambda n: (0, 0)),
            pl.BlockSpec((51, 101), lambda n: (0, 0)),
            pl.BlockSpec((25, 49), lambda n: (0, 0)),
        ],
        out_specs=pl.BlockSpec((1, 6, 25, 256), lambda n: (n, 0, 0, 0)),
        compiler_params=pltpu.CompilerParams(
            dimension_semantics=("parallel",)),
    )(patches1, c1p, w2r, w3r, w4r, w5r, sel1, sel2, sel3)

    flat = feats.transpose(0, 3, 1, 2).reshape(N, 38400)

    tn, tk = 768, 6400
    h1 = pl.pallas_call(
        _fc1_kernel,
        out_shape=jax.ShapeDtypeStruct((N, 1536), jnp.bfloat16),
        grid=(1536 // tn, 38400 // tk),
        in_specs=[
            pl.BlockSpec((N, tk), lambda j, k: (0, k)),
            pl.BlockSpec((tk, tn), lambda j, k: (k, j)),
            pl.BlockSpec((1, tn), lambda j, k: (0, j)),
        ],
        out_specs=pl.BlockSpec((N, tn), lambda j, k: (0, j)),
        scratch_shapes=[pltpu.VMEM((N, tn), jnp.float32)],
        compiler_params=pltpu.CompilerParams(
            dimension_semantics=("parallel", "arbitrary")),
    )(flat, l1_w, l1_b.reshape(1, 1536).astype(jnp.float32))

    w3p = jnp.pad(l3_w, ((0, 0), (0, 118)))
    b3p = jnp.pad(l3_b, (0, 118)).reshape(1, 128).astype(jnp.float32)
    out = pl.pallas_call(
        _head_kernel,
        out_shape=jax.ShapeDtypeStruct((N, 128), jnp.float32),
        grid=(1,),
        in_specs=[
            pl.BlockSpec((N, 1536), lambda i: (0, 0)),
            pl.BlockSpec((1536, 1536), lambda i: (0, 0)),
            pl.BlockSpec((1, 1536), lambda i: (0, 0)),
            pl.BlockSpec((1536, 128), lambda i: (0, 0)),
            pl.BlockSpec((1, 128), lambda i: (0, 0)),
        ],
        out_specs=pl.BlockSpec((N, 128), lambda i: (0, 0)),
    )(h1, l2_w, l2_b.reshape(1, 1536).astype(jnp.float32), w3p, b3p)
    return out[:, :10]


# speedup vs baseline: 1.6345x; 1.6345x over previous
import functools

import jax
import numpy as np
import jax.numpy as jnp
from jax.experimental import pallas as pl
from jax.experimental.pallas import tpu as pltpu


def _maxpool3x3s2(a, sel):
    H, W, C = a.shape
    Ho, Wo = (H - 3) // 2 + 1, (W - 3) // 2 + 1
    v = _relu_bf16(a)
    Hp = ((H + 1) // 2) * 2
    if Hp != H:
        v = jnp.concatenate([v, jnp.zeros((Hp - H, W, C), v.dtype)], axis=0)
    v = v.reshape(Hp // 2, 2, W, C)
    e0, e1 = v[:, 0], v[:, 1]
    rm = jnp.maximum(jnp.maximum(e0[:Ho], e1[:Ho]), e0[1:Ho + 1])
    sm = jnp.maximum(
        jnp.maximum(rm[:, 0:W - 2], rm[:, 1:W - 1]), rm[:, 2:W])
    selb = jnp.broadcast_to(sel[None], (Ho, Wo, W - 2))
    out = jax.lax.dot_general(
        selb, sm, (((2,), (1,)), ((0,), (0,))),
        preferred_element_type=jnp.float32)
    return out.astype(jnp.bfloat16)


def _zpad(v, p):
    H, W, C = v.shape
    v = jnp.concatenate(
        [jnp.zeros((H, p, C), v.dtype), v, jnp.zeros((H, p, C), v.dtype)], axis=1)
    Wp = W + 2 * p
    v = jnp.concatenate(
        [jnp.zeros((p, Wp, C), v.dtype), v, jnp.zeros((p, Wp, C), v.dtype)], axis=0)
    return v


def _rowcat(v, kw, wo):
    return jnp.concatenate([v[:, dw:dw + wo, :] for dw in range(kw)], axis=-1)


def _conv_taps(xw, wr, ho, kh):
    acc = None
    for dh in range(kh):
        a = jax.lax.dot_general(
            xw[dh:dh + ho], wr[dh],
            (((2,), (0,)), ((), ())), preferred_element_type=jnp.float32)
        acc = a if acc is None else acc + a
    return acc


def _relu_bf16(a):
    return jnp.maximum(a, 0.0).astype(jnp.bfloat16)


def _net_kernel(p1_ref, w1_ref, w2_ref, w3_ref, w4_ref, w5_ref,
                sel1_ref, sel2_ref, sel3_ref, o_ref):
    a = jax.lax.dot_general(
        p1_ref[0], w1_ref[...],
        (((2,), (0,)), ((), ())), preferred_element_type=jnp.float32)
    x = _maxpool3x3s2(a, sel1_ref[...])

    xw = _rowcat(_zpad(x, 2), 5, 103)
    x = _maxpool3x3s2(_conv_taps(xw, w2_ref[...], 27, 5),
                      sel2_ref[...])

    xw = _rowcat(_zpad(x, 1), 3, 51)
    x = _relu_bf16(_conv_taps(xw, w3_ref[...], 13, 3))

    xw = _rowcat(_zpad(x, 1), 3, 51)
    x = _relu_bf16(_conv_taps(xw, w4_ref[...], 13, 3))

    xw = _rowcat(_zpad(x, 1), 3, 51)
    o_ref[0] = _maxpool3x3s2(_conv_taps(xw, w5_ref[...], 13, 3),
                             sel3_ref[...])


def _fc1_kernel(a_ref, w_ref, b_ref, o_ref, acc_ref):
    k = pl.program_id(1)

    @pl.when(k == 0)
    def _():
        acc_ref[...] = jnp.zeros_like(acc_ref)

    acc_ref[...] += jnp.dot(a_ref[...], w_ref[...],
                            preferred_element_type=jnp.float32)

    @pl.when(k == pl.num_programs(1) - 1)
    def _():
        o_ref[...] = _relu_bf16(acc_ref[...] + b_ref[...])


def _head_kernel(a_ref, w2_ref, b2_ref, w3_ref, b3_ref, o_ref):
    h = jnp.dot(a_ref[...], w2_ref[...], preferred_element_type=jnp.float32)
    h = _relu_bf16(h + b2_ref[...])
    z = jnp.dot(h, w3_ref[...], preferred_element_type=jnp.float32)
    z = jnp.maximum(z + b3_ref[...], 0.0)
    col = jax.lax.broadcasted_iota(jnp.int32, z.shape, 1)
    mask = col < 10
    zm = jnp.where(mask, z, -jnp.inf)
    m = jnp.max(zm, axis=1, keepdims=True)
    e = jnp.where(mask, jnp.exp(zm - m), 0.0)
    s = jnp.sum(e, axis=1, keepdims=True)
    o_ref[...] = e / s


def _forward(x, c1, c2, c3, c4, c5, l1_w, l1_b, l2_w, l2_b, l3_w, l3_b):
    N = x.shape[0]
    xb = x[:, 0].astype(jnp.bfloat16)

    patches1 = jax.lax.conv_general_dilated_patches(
        xb[..., None], (11, 11), (2, 2), "VALID",
        dimension_numbers=("NHWC", "HWIO", "NHWC"))

    w2r = c2.reshape(96, 5, 5, 256).transpose(1, 2, 0, 3).reshape(5, 480, 256)
    w3r = c3.reshape(256, 3, 3, 384).transpose(1, 2, 0, 3).reshape(3, 768, 384)
    w4r = c4.reshape(384, 3, 3, 384).transpose(1, 2, 0, 3).reshape(3, 1152, 384)
    w5r = c5.reshape(384, 3, 3, 256).transpose(1, 2, 0, 3).reshape(3, 1152, 256)

    def _sel(wo, wm):
        return (jnp.arange(wo)[:, None] * 2
                == jnp.arange(wm)[None, :]).astype(jnp.bfloat16)
    sel1, sel2, sel3 = _sel(103, 205), _sel(51, 101), _sel(25, 49)

    feats = pl.pallas_call(
        _net_kernel,
        out_shape=jax.ShapeDtypeStruct((N, 6, 25, 256), jnp.bfloat16),
        grid=(N,),
        in_specs=[
            pl.BlockSpec((1, 55, 207, 121), lambda n: (n, 0, 0, 0)),
            pl.BlockSpec((121, 96), lambda n: (0, 0)),
            pl.BlockSpec((5, 480, 256), lambda n: (0, 0, 0)),
            pl.BlockSpec((3, 768, 384), lambda n: (0, 0, 0)),
            pl.BlockSpec((3, 1152, 384), lambda n: (0, 0, 0)),
            pl.BlockSpec((3, 1152, 256), lambda n: (0, 0, 0)),
            pl.BlockSpec((103, 205), lambda n: (0, 0)),
            pl.BlockSpec((51, 101), lambda n: (0, 0)),
            pl.BlockSpec((25, 49), lambda n: (0, 0)),
        ],
        out_specs=pl.BlockSpec((1, 6, 25, 256), lambda n: (n, 0, 0, 0)),
        compiler_params=pltpu.CompilerParams(
            dimension_semantics=("parallel",)),
    )(patches1, c1, w2r, w3r, w4r, w5r, sel1, sel2, sel3)

    flat = feats.transpose(0, 3, 1, 2).reshape(N, 38400)

    tn, tk = 768, 6400
    h1 = pl.pallas_call(
        _fc1_kernel,
        out_shape=jax.ShapeDtypeStruct((N, 1536), jnp.bfloat16),
        grid=(1536 // tn, 38400 // tk),
        in_specs=[
            pl.BlockSpec((N, tk), lambda j, k: (0, k)),
            pl.BlockSpec((tk, tn), lambda j, k: (k, j)),
            pl.BlockSpec((1, tn), lambda j, k: (0, j)),
        ],
        out_specs=pl.BlockSpec((N, tn), lambda j, k: (0, j)),
        scratch_shapes=[pltpu.VMEM((N, tn), jnp.float32)],
        compiler_params=pltpu.CompilerParams(
            dimension_semantics=("parallel", "arbitrary")),
    )(flat, l1_w, l1_b.reshape(1, 1536).astype(jnp.float32))

    w3p = jnp.pad(l3_w, ((0, 0), (0, 118)))
    b3p = jnp.pad(l3_b, (0, 118)).reshape(1, 128).astype(jnp.float32)
    out = pl.pallas_call(
        _head_kernel,
        out_shape=jax.ShapeDtypeStruct((N, 128), jnp.float32),
        grid=(1,),
        in_specs=[
            pl.BlockSpec((N, 1536), lambda i: (0, 0)),
            pl.BlockSpec((1536, 1536), lambda i: (0, 0)),
            pl.BlockSpec((1, 1536), lambda i: (0, 0)),
            pl.BlockSpec((1536, 128), lambda i: (0, 0)),
            pl.BlockSpec((1, 128), lambda i: (0, 0)),
        ],
        out_specs=pl.BlockSpec((N, 128), lambda i: (0, 0)),
    )(h1, l2_w, l2_b.reshape(1, 1536).astype(jnp.float32), w3p, b3p)
    return out[:, :10]


def kernel(x, c1, c2, c3, c4, c5, l1_w, l1_b, l2_w, l2_b, l3_w, l3_b):
    devs = jax.devices()
    nd = 2 if len(devs) >= 2 and x.shape[0] % 2 == 0 else 1
    mesh = jax.sharding.Mesh(np.array(devs[:nd]), ("b",))
    P = jax.sharding.PartitionSpec
    fwd = jax.shard_map(
        _forward, mesh=mesh,
        in_specs=(P("b"),) + (P(),) * 11,
        out_specs=P("b"), check_vma=False)
    return fwd(x, c1, c2, c3, c4, c5, l1_w, l1_b, l2_w, l2_b, l3_w, l3_b)


# speedup vs baseline: 1.6746x; 1.0245x over previous
import jax
import numpy as np
import jax.numpy as jnp
from jax.experimental import pallas as pl
from jax.experimental.pallas import tpu as pltpu


def _maxpool3x3s2(a, sel):
    H, W, C = a.shape
    Ho, Wo = (H - 3) // 2 + 1, (W - 3) // 2 + 1
    v = _relu_bf16(a)
    Hp = ((H + 1) // 2) * 2
    if Hp != H:
        v = jnp.concatenate([v, jnp.zeros((Hp - H, W, C), v.dtype)], axis=0)
    v = v.reshape(Hp // 2, 2, W, C)
    e0, e1 = v[:, 0], v[:, 1]
    rm = jnp.maximum(jnp.maximum(e0[:Ho], e1[:Ho]), e0[1:Ho + 1])
    sm = jnp.maximum(
        jnp.maximum(rm[:, 0:W - 2], rm[:, 1:W - 1]), rm[:, 2:W])
    selb = jnp.broadcast_to(sel[None], (Ho, Wo, W - 2))
    out = jax.lax.dot_general(
        selb, sm, (((2,), (1,)), ((0,), (0,))),
        preferred_element_type=jnp.float32)
    return out.astype(jnp.bfloat16)


def _zpad(v, p):
    H, W, C = v.shape
    v = jnp.concatenate(
        [jnp.zeros((H, p, C), v.dtype), v, jnp.zeros((H, p, C), v.dtype)], axis=1)
    Wp = W + 2 * p
    v = jnp.concatenate(
        [jnp.zeros((p, Wp, C), v.dtype), v, jnp.zeros((p, Wp, C), v.dtype)], axis=0)
    return v


def _rowcat(v, kw, wo):
    return jnp.concatenate([v[:, dw:dw + wo, :] for dw in range(kw)], axis=-1)


def _conv_taps(xw, wr, ho, kh):
    acc = None
    for dh in range(kh):
        a = jax.lax.dot_general(
            xw[dh:dh + ho], wr[dh],
            (((2,), (0,)), ((), ())), preferred_element_type=jnp.float32)
        acc = a if acc is None else acc + a
    return acc


def _relu_bf16(a):
    return jnp.maximum(a, 0.0).astype(jnp.bfloat16)


def _net_kernel(p1_ref, w1_ref, w2_ref, w3_ref, w4_ref, w5_ref,
                sel1_ref, sel2_ref, sel3_ref, o_ref):
    a = jax.lax.dot_general(
        p1_ref[0], w1_ref[...],
        (((2,), (0,)), ((), ())), preferred_element_type=jnp.float32)
    x = _maxpool3x3s2(a, sel1_ref[...])

    xw = _rowcat(_zpad(x, 2), 5, 103)
    x = _maxpool3x3s2(_conv_taps(xw, w2_ref[...], 27, 5),
                      sel2_ref[...])

    xw = _rowcat(_zpad(x, 1), 3, 51)
    x = _relu_bf16(_conv_taps(xw, w3_ref[...], 13, 3))

    xw = _rowcat(_zpad(x, 1), 3, 51)
    x = _relu_bf16(_conv_taps(xw, w4_ref[...], 13, 3))

    xw = _rowcat(_zpad(x, 1), 3, 51)
    o_ref[0] = _maxpool3x3s2(_conv_taps(xw, w5_ref[...], 13, 3),
                             sel3_ref[...])


def _fc1_kernel(a_ref, w_ref, b_ref, o_ref, acc_ref):
    k = pl.program_id(1)

    @pl.when(k == 0)
    def _():
        acc_ref[...] = jnp.zeros_like(acc_ref)

    acc_ref[...] += jnp.dot(a_ref[...], w_ref[...],
                            preferred_element_type=jnp.float32)

    @pl.when(k == pl.num_programs(1) - 1)
    def _():
        o_ref[...] = _relu_bf16(acc_ref[...] + b_ref[...])


def _head_kernel(a_ref, w2_ref, b2_ref, w3_ref, b3_ref, o_ref):
    h = jnp.dot(a_ref[...], w2_ref[...], preferred_element_type=jnp.float32)
    h = _relu_bf16(h + b2_ref[...])
    z = jnp.dot(h, w3_ref[...], preferred_element_type=jnp.float32)
    z = jnp.maximum(z + b3_ref[...], 0.0)
    col = jax.lax.broadcasted_iota(jnp.int32, z.shape, 1)
    mask = col < 10
    zm = jnp.where(mask, z, -jnp.inf)
    m = jnp.max(zm, axis=1, keepdims=True)
    e = jnp.where(mask, jnp.exp(zm - m), 0.0)
    s = jnp.sum(e, axis=1, keepdims=True)
    o_ref[...] = e / s


def _forward(x, c1, c2, c3, c4, c5, l1_w, l1_b, l2_w, l2_b, l3_w, l3_b):
    N = x.shape[0]
    xb = x[:, 0].astype(jnp.bfloat16)

    patches1 = jax.lax.conv_general_dilated_patches(
        xb[..., None], (11, 11), (2, 2), "VALID",
        dimension_numbers=("NHWC", "HWIO", "NHWC"))

    w2r = c2.reshape(96, 5, 5, 256).transpose(1, 2, 0, 3).reshape(5, 480, 256)
    w3r = c3.reshape(256, 3, 3, 384).transpose(1, 2, 0, 3).reshape(3, 768, 384)
    w4r = c4.reshape(384, 3, 3, 384).transpose(1, 2, 0, 3).reshape(3, 1152, 384)
    w5r = c5.reshape(384, 3, 3, 256).transpose(1, 2, 0, 3).reshape(3, 1152, 256)

    def _sel(wo, wm):
        return (jnp.arange(wo)[:, None] * 2
                == jnp.arange(wm)[None, :]).astype(jnp.bfloat16)
    sel1, sel2, sel3 = _sel(103, 205), _sel(51, 101), _sel(25, 49)

    feats = pl.pallas_call(
        _net_kernel,
        out_shape=jax.ShapeDtypeStruct((N, 6, 25, 256), jnp.bfloat16),
        grid=(N,),
        in_specs=[
            pl.BlockSpec((1, 55, 207, 121), lambda n: (n, 0, 0, 0)),
            pl.BlockSpec((121, 96), lambda n: (0, 0)),
            pl.BlockSpec((5, 480, 256), lambda n: (0, 0, 0)),
            pl.BlockSpec((3, 768, 384), lambda n: (0, 0, 0)),
            pl.BlockSpec((3, 1152, 384), lambda n: (0, 0, 0)),
            pl.BlockSpec((3, 1152, 256), lambda n: (0, 0, 0)),
            pl.BlockSpec((103, 205), lambda n: (0, 0)),
            pl.BlockSpec((51, 101), lambda n: (0, 0)),
            pl.BlockSpec((25, 49), lambda n: (0, 0)),
        ],
        out_specs=pl.BlockSpec((1, 6, 25, 256), lambda n: (n, 0, 0, 0)),
        compiler_params=pltpu.CompilerParams(
            dimension_semantics=("parallel",)),
    )(patches1, c1, w2r, w3r, w4r, w5r, sel1, sel2, sel3)

    flat = feats.transpose(0, 3, 1, 2).reshape(N, 38400)

    tn, tk = 768, 6400
    h1 = pl.pallas_call(
        _fc1_kernel,
        out_shape=jax.ShapeDtypeStruct((N, 1536), jnp.bfloat16),
        grid=(1536 // tn, 38400 // tk),
        in_specs=[
            pl.BlockSpec((N, tk), lambda j, k: (0, k)),
            pl.BlockSpec((tk, tn), lambda j, k: (k, j)),
            pl.BlockSpec((1, tn), lambda j, k: (0, j)),
        ],
        out_specs=pl.BlockSpec((N, tn), lambda j, k: (0, j)),
        scratch_shapes=[pltpu.VMEM((N, tn), jnp.float32)],
        compiler_params=pltpu.CompilerParams(
            dimension_semantics=("parallel", "arbitrary")),
    )(flat, l1_w, l1_b.reshape(1, 1536).astype(jnp.float32))

    w3p = jnp.pad(l3_w, ((0, 0), (0, 118)))
    b3p = jnp.pad(l3_b, (0, 118)).reshape(1, 128).astype(jnp.float32)
    out = pl.pallas_call(
        _head_kernel,
        out_shape=jax.ShapeDtypeStruct((N, 128), jnp.float32),
        grid=(1,),
        in_specs=[
            pl.BlockSpec((N, 1536), lambda i: (0, 0)),
            pl.BlockSpec((1536, 1536), lambda i: (0, 0)),
            pl.BlockSpec((1, 1536), lambda i: (0, 0)),
            pl.BlockSpec((1536, 128), lambda i: (0, 0)),
            pl.BlockSpec((1, 128), lambda i: (0, 0)),
        ],
        out_specs=pl.BlockSpec((N, 128), lambda i: (0, 0)),
    )(h1, l2_w, l2_b.reshape(1, 1536).astype(jnp.float32), w3p, b3p)
    return out[:, :10]


def kernel(x, c1, c2, c3, c4, c5, l1_w, l1_b, l2_w, l2_b, l3_w, l3_b):
    devs = jax.devices()
    nd = 2 if len(devs) >= 2 and x.shape[0] % 2 == 0 else 1
    mesh = jax.sharding.Mesh(np.array(devs[:nd]), ("b",))
    P = jax.sharding.PartitionSpec
    fwd = jax.shard_map(
        _forward, mesh=mesh,
        in_specs=(P("b"),) + (P(),) * 11,
        out_specs=P("b"), check_vma=False)
    return fwd(x, c1, c2, c3, c4, c5, l1_w, l1_b, l2_w, l2_b, l3_w, l3_b)
